# baseline (device time: 421759 ns/iter reference)
import jax
import jax.numpy as jnp
from jax import lax
from jax.experimental import pallas as pl
from jax.experimental.pallas import tpu as pltpu

N_DEV = 16


def kernel(x, w_mat, scale_x, scale_w):
    m_glob, k_loc = x.shape
    _, n = w_mat.shape
    m_chunk = m_glob // N_DEV

    def body(x_ref, w_ref, sx_ref, sw_ref, out_ref,
             buf_ref, w_bf16_ref, send_sems, recv_sems, credit_sem):
        my = lax.axis_index("i")
        left = lax.rem(my - 1 + N_DEV, N_DEV)
        right = lax.rem(my + 1, N_DEV)

        barrier = pltpu.get_barrier_semaphore()
        for nbr in (left, right):
            pl.semaphore_signal(
                barrier, inc=1,
                device_id=(nbr,), device_id_type=pl.DeviceIdType.MESH,
            )
        pl.semaphore_wait(barrier, 2)

        w_bf16_ref[...] = w_ref[...].astype(jnp.bfloat16)

        def chunk(idx):
            xa = x_ref[pl.ds(idx * m_chunk, m_chunk), :].astype(jnp.bfloat16)
            return lax.dot_general(
                xa, w_bf16_ref[...], (((1,), (0,)), ((), ())),
                preferred_element_type=jnp.float32,
            )

        buf_ref[0] = chunk(lax.rem(my - 1 + N_DEV, N_DEV))

        for h in range(N_DEV - 1):
            s = h % 2
            r = (h + 1) % 2
            if h > 0:
                pl.semaphore_wait(credit_sem, 1)
            rdma = pltpu.make_async_remote_copy(
                src_ref=buf_ref.at[s],
                dst_ref=buf_ref.at[r],
                send_sem=send_sems.at[s],
                recv_sem=recv_sems.at[r],
                device_id=(right,),
                device_id_type=pl.DeviceIdType.MESH,
            )
            rdma.start()
            rdma.wait()
            if h < N_DEV - 2:
                pl.semaphore_signal(
                    credit_sem, inc=1,
                    device_id=(left,), device_id_type=pl.DeviceIdType.MESH,
                )
            idx = lax.rem(my - 2 - h + 2 * N_DEV, N_DEV)
            buf_ref[r] = buf_ref[r] + chunk(idx)

        alpha = sx_ref[0] * sw_ref[0]
        y = buf_ref[(N_DEV - 1) % 2] * alpha
        yc = jnp.clip(y, -60.0, 60.0)
        out_ref[...] = y / (1.0 + jnp.exp(-yc))

    return pl.pallas_call(
        body,
        out_shape=jax.ShapeDtypeStruct((m_chunk, n), jnp.float32),
        in_specs=[
            pl.BlockSpec(memory_space=pltpu.VMEM),
            pl.BlockSpec(memory_space=pltpu.VMEM),
            pl.BlockSpec(memory_space=pltpu.SMEM),
            pl.BlockSpec(memory_space=pltpu.SMEM),
        ],
        out_specs=pl.BlockSpec(memory_space=pltpu.VMEM),
        scratch_shapes=[
            pltpu.VMEM((2, m_chunk, n), jnp.float32),
            pltpu.VMEM((k_loc, n), jnp.bfloat16),
            pltpu.SemaphoreType.DMA((2,)),
            pltpu.SemaphoreType.DMA((2,)),
            pltpu.SemaphoreType.REGULAR,
        ],
        compiler_params=pltpu.CompilerParams(collective_id=0),
    )(x, w_mat, scale_x, scale_w)


# device time: 225790 ns/iter; 1.8679x vs baseline; 1.8679x over previous
import jax
import jax.numpy as jnp
from jax import lax
from jax.experimental import pallas as pl
from jax.experimental.pallas import tpu as pltpu

N_DEV = 16
N_STREAMS = 4
S = 3


def kernel(x, w_mat, scale_x, scale_w):
    m_glob, k_loc = x.shape
    _, n = w_mat.shape
    m_chunk = m_glob // N_DEV
    n_half = n // 2
    n_sub = n // N_STREAMS

    def body(x_ref, w_ref, sx_ref, sw_ref, out_ref,
             buf_ref, w_bf16_ref, tmp_ref, send_sems, recv_sems, credit_sems):
        my = lax.axis_index("i")
        left = lax.rem(my - 1 + N_DEV, N_DEV)
        right = lax.rem(my + 1, N_DEV)

        peer_out = [right, right, left, left]
        peer_in = [left, left, right, right]

        barrier = pltpu.get_barrier_semaphore()
        for nbr in (left, right):
            pl.semaphore_signal(
                barrier, inc=1,
                device_id=(nbr,), device_id_type=pl.DeviceIdType.MESH,
            )
        pl.semaphore_wait(barrier, 2)

        w_bf16_ref[...] = w_ref[...].astype(jnp.bfloat16)

        def dot_chunk(idx, col0, ncol):
            xa = x_ref[pl.ds(idx * m_chunk, m_chunk), :].astype(jnp.bfloat16)
            return lax.dot_general(
                xa, w_bf16_ref[:, col0:col0 + ncol], (((1,), (0,)), ((), ())),
                preferred_element_type=jnp.float32,
            )

        def mod(v):
            return lax.rem(v + 2 * N_DEV, N_DEV)

        buf_ref[0, 0] = dot_chunk(mod(my - 1), 0 * n_sub, n_sub)
        buf_ref[1, 0] = dot_chunk(mod(my - 1), 1 * n_sub, n_sub)
        buf_ref[2, 0] = dot_chunk(mod(my + 1), 2 * n_sub, n_sub)
        buf_ref[3, 0] = dot_chunk(mod(my + 1), 3 * n_sub, n_sub)

        order = (0, 2, 1, 3)

        for h in range(N_DEV - 1):
            s = h % S
            r = (h + 1) % S

            rdmas = []
            for k in range(N_STREAMS):
                if h >= S - 1:
                    pl.semaphore_wait(credit_sems.at[k], 1)
                rdmas.append(pltpu.make_async_remote_copy(
                    src_ref=buf_ref.at[k, s],
                    dst_ref=buf_ref.at[k, r],
                    send_sem=send_sems.at[k, s],
                    recv_sem=recv_sems.at[k, r],
                    device_id=(peer_out[k],),
                    device_id_type=pl.DeviceIdType.MESH,
                ))
            for k in order:
                rdmas[k].start()

            idx_r = mod(my - 2 - h)
            idx_l = mod(my + 2 + h)
            tmp_ref[:, :n_half] = dot_chunk(idx_r, 0, n_half)
            tmp_ref[:, n_half:] = dot_chunk(idx_l, n_half, n_half)

            for k in order:
                rdmas[k].wait_recv()
                buf_ref[k, r] = (
                    buf_ref[k, r] + tmp_ref[:, k * n_sub:(k + 1) * n_sub]
                )
                rdmas[k].wait_send()
                if h < N_DEV - S:
                    pl.semaphore_signal(
                        credit_sems.at[k], inc=1,
                        device_id=(peer_in[k],),
                        device_id_type=pl.DeviceIdType.MESH,
                    )

        fslot = (N_DEV - 1) % S
        alpha = sx_ref[0] * sw_ref[0]
        for k in range(N_STREAMS):
            y = buf_ref[k, fslot] * alpha
            yc = jnp.clip(y, -60.0, 60.0)
            out_ref[:, k * n_sub:(k + 1) * n_sub] = y / (1.0 + jnp.exp(-yc))

    return pl.pallas_call(
        body,
        out_shape=jax.ShapeDtypeStruct((m_chunk, n), jnp.float32),
        in_specs=[
            pl.BlockSpec(memory_space=pltpu.VMEM),
            pl.BlockSpec(memory_space=pltpu.VMEM),
            pl.BlockSpec(memory_space=pltpu.SMEM),
            pl.BlockSpec(memory_space=pltpu.SMEM),
        ],
        out_specs=pl.BlockSpec(memory_space=pltpu.VMEM),
        scratch_shapes=[
            pltpu.VMEM((N_STREAMS, S, m_chunk, n_sub), jnp.float32),
            pltpu.VMEM((k_loc, n), jnp.bfloat16),
            pltpu.VMEM((m_chunk, n), jnp.float32),
            pltpu.SemaphoreType.DMA((N_STREAMS, S)),
            pltpu.SemaphoreType.DMA((N_STREAMS, S)),
            pltpu.SemaphoreType.REGULAR((N_STREAMS,)),
        ],
        compiler_params=pltpu.CompilerParams(collective_id=0),
    )(x, w_mat, scale_x, scale_w)


# device time: 181202 ns/iter; 2.3276x vs baseline; 1.2461x over previous
import jax
import jax.numpy as jnp
from jax import lax
from jax.experimental import pallas as pl
from jax.experimental.pallas import tpu as pltpu

N_DEV = 16
N_STREAMS = 4
S = 3


def kernel(x, w_mat, scale_x, scale_w):
    m_glob, k_loc = x.shape
    _, n = w_mat.shape
    m_chunk = m_glob // N_DEV
    n_half = n // 2
    n_sub = n // N_STREAMS

    def body(x_ref, w_ref, sx_ref, sw_ref, out_ref,
             buf_ref, w_bf16_ref, tmp_ref, send_sems, recv_sems, credit_sems):
        my = lax.axis_index("i")
        left = lax.rem(my - 1 + N_DEV, N_DEV)
        right = lax.rem(my + 1, N_DEV)

        peer_out = [right, right, left, left]
        peer_in = [left, left, right, right]

        barrier = pltpu.get_barrier_semaphore()
        for nbr in (left, right):
            pl.semaphore_signal(
                barrier, inc=1,
                device_id=(nbr,), device_id_type=pl.DeviceIdType.MESH,
            )
        pl.semaphore_wait(barrier, 2)

        w_bf16_ref[...] = w_ref[...].astype(jnp.bfloat16)

        def dot_chunk(idx, col0, ncol):
            xa = x_ref[pl.ds(idx * m_chunk, m_chunk), :].astype(jnp.bfloat16)
            return lax.dot_general(
                xa, w_bf16_ref[:, col0:col0 + ncol], (((1,), (0,)), ((), ())),
                preferred_element_type=jnp.float32,
            )

        def mod(v):
            return lax.rem(v + 2 * N_DEV, N_DEV)

        buf_ref[0, 0] = dot_chunk(mod(my - 1), 0 * n_sub, n_sub)
        buf_ref[1, 0] = dot_chunk(mod(my - 1), 1 * n_sub, n_sub)
        buf_ref[2, 0] = dot_chunk(mod(my + 1), 2 * n_sub, n_sub)
        buf_ref[3, 0] = dot_chunk(mod(my + 1), 3 * n_sub, n_sub)

        order = (0, 2, 1, 3)
        alpha = sx_ref[0] * sw_ref[0]

        def make_rdma(k, h):
            return pltpu.make_async_remote_copy(
                src_ref=buf_ref.at[k, h % S],
                dst_ref=buf_ref.at[k, (h + 1) % S],
                send_sem=send_sems.at[k, h % S],
                recv_sem=recv_sems.at[k, (h + 1) % S],
                device_id=(peer_out[k],),
                device_id_type=pl.DeviceIdType.MESH,
            )

        rdmas = [make_rdma(k, 0) for k in range(N_STREAMS)]
        for k in order:
            rdmas[k].start()

        for h in range(N_DEV - 1):
            r = (h + 1) % S

            idx_r = mod(my - 2 - h)
            idx_l = mod(my + 2 + h)
            tmp_ref[:, :n_half] = dot_chunk(idx_r, 0, n_half)
            tmp_ref[:, n_half:] = dot_chunk(idx_l, n_half, n_half)

            for k in order:
                cols = pl.ds(k * n_sub, n_sub)
                rdmas[k].wait_recv()
                if h < N_DEV - 2:
                    buf_ref[k, r] = buf_ref[k, r] + tmp_ref[:, cols]
                    if h + 1 >= S - 1:
                        pl.semaphore_wait(credit_sems.at[k], 1)
                    nxt = make_rdma(k, h + 1)
                    nxt.start()
                else:
                    y = (buf_ref[k, r] + tmp_ref[:, cols]) * alpha
                    yc = jnp.clip(y, -60.0, 60.0)
                    out_ref[:, cols] = y / (1.0 + jnp.exp(-yc))
                rdmas[k].wait_send()
                if h < N_DEV - S:
                    pl.semaphore_signal(
                        credit_sems.at[k], inc=1,
                        device_id=(peer_in[k],),
                        device_id_type=pl.DeviceIdType.MESH,
                    )
                if h < N_DEV - 2:
                    rdmas[k] = nxt

    return pl.pallas_call(
        body,
        out_shape=jax.ShapeDtypeStruct((m_chunk, n), jnp.float32),
        in_specs=[
            pl.BlockSpec(memory_space=pltpu.VMEM),
            pl.BlockSpec(memory_space=pltpu.VMEM),
            pl.BlockSpec(memory_space=pltpu.SMEM),
            pl.BlockSpec(memory_space=pltpu.SMEM),
        ],
        out_specs=pl.BlockSpec(memory_space=pltpu.VMEM),
        scratch_shapes=[
            pltpu.VMEM((N_STREAMS, S, m_chunk, n_sub), jnp.float32),
            pltpu.VMEM((k_loc, n), jnp.bfloat16),
            pltpu.VMEM((m_chunk, n), jnp.float32),
            pltpu.SemaphoreType.DMA((N_STREAMS, S)),
            pltpu.SemaphoreType.DMA((N_STREAMS, S)),
            pltpu.SemaphoreType.REGULAR((N_STREAMS,)),
        ],
        compiler_params=pltpu.CompilerParams(collective_id=0),
    )(x, w_mat, scale_x, scale_w)


# device time: 180952 ns/iter; 2.3308x vs baseline; 1.0014x over previous
import jax
import jax.numpy as jnp
from jax import lax
from jax.experimental import pallas as pl
from jax.experimental.pallas import tpu as pltpu

N_DEV = 16
N_STREAMS = 4
S = 3


def kernel(x, w_mat, scale_x, scale_w):
    m_glob, k_loc = x.shape
    _, n = w_mat.shape
    m_chunk = m_glob // N_DEV
    n_half = n // 2
    n_sub = n // N_STREAMS

    def body(x_ref, w_ref, sx_ref, sw_ref, out_ref,
             buf_ref, w_bf16_ref, tmp_ref, send_sems, recv_sems, credit_sems):
        my = lax.axis_index("i")
        left = lax.rem(my - 1 + N_DEV, N_DEV)
        right = lax.rem(my + 1, N_DEV)

        peer_out = [right, right, left, left]
        peer_in = [left, left, right, right]

        barrier = pltpu.get_barrier_semaphore()
        for nbr in (left, right):
            pl.semaphore_signal(
                barrier, inc=1,
                device_id=(nbr,), device_id_type=pl.DeviceIdType.MESH,
            )
        pl.semaphore_wait(barrier, 2)

        w_bf16_ref[...] = w_ref[...].astype(jnp.bfloat16)

        def dot_chunk(idx, col0, ncol):
            xa = x_ref[pl.ds(idx * m_chunk, m_chunk), :].astype(jnp.bfloat16)
            return lax.dot_general(
                xa, w_bf16_ref[:, col0:col0 + ncol], (((1,), (0,)), ((), ())),
                preferred_element_type=jnp.float32,
            )

        def mod(v):
            return lax.rem(v + 2 * N_DEV, N_DEV)

        init_idx = [mod(my - 1), mod(my - 1), mod(my + 1), mod(my + 1)]

        order = (0, 2, 1, 3)
        alpha = sx_ref[0] * sw_ref[0]

        def make_rdma(k, h):
            return pltpu.make_async_remote_copy(
                src_ref=buf_ref.at[k, h % S],
                dst_ref=buf_ref.at[k, (h + 1) % S],
                send_sem=send_sems.at[k, h % S],
                recv_sem=recv_sems.at[k, (h + 1) % S],
                device_id=(peer_out[k],),
                device_id_type=pl.DeviceIdType.MESH,
            )

        rdmas = [make_rdma(k, 0) for k in range(N_STREAMS)]
        for k in order:
            buf_ref[k, 0] = dot_chunk(init_idx[k], k * n_sub, n_sub)
            rdmas[k].start()

        for h in range(N_DEV - 1):
            r = (h + 1) % S

            idx_r = mod(my - 2 - h)
            idx_l = mod(my + 2 + h)
            tmp_ref[:, :n_half] = dot_chunk(idx_r, 0, n_half)
            tmp_ref[:, n_half:] = dot_chunk(idx_l, n_half, n_half)

            for k in order:
                cols = pl.ds(k * n_sub, n_sub)
                rdmas[k].wait_recv()
                if h < N_DEV - 2:
                    buf_ref[k, r] = buf_ref[k, r] + tmp_ref[:, cols]
                    if h + 1 >= S - 1:
                        pl.semaphore_wait(credit_sems.at[k], 1)
                    nxt = make_rdma(k, h + 1)
                    nxt.start()
                else:
                    y = (buf_ref[k, r] + tmp_ref[:, cols]) * alpha
                    yc = jnp.clip(y, -60.0, 60.0)
                    out_ref[:, cols] = y / (1.0 + jnp.exp(-yc))
                rdmas[k].wait_send()
                if h < N_DEV - S:
                    pl.semaphore_signal(
                        credit_sems.at[k], inc=1,
                        device_id=(peer_in[k],),
                        device_id_type=pl.DeviceIdType.MESH,
                    )
                if h < N_DEV - 2:
                    rdmas[k] = nxt

    return pl.pallas_call(
        body,
        out_shape=jax.ShapeDtypeStruct((m_chunk, n), jnp.float32),
        in_specs=[
            pl.BlockSpec(memory_space=pltpu.VMEM),
            pl.BlockSpec(memory_space=pltpu.VMEM),
            pl.BlockSpec(memory_space=pltpu.SMEM),
            pl.BlockSpec(memory_space=pltpu.SMEM),
        ],
        out_specs=pl.BlockSpec(memory_space=pltpu.VMEM),
        scratch_shapes=[
            pltpu.VMEM((N_STREAMS, S, m_chunk, n_sub), jnp.float32),
            pltpu.VMEM((k_loc, n), jnp.bfloat16),
            pltpu.VMEM((m_chunk, n), jnp.float32),
            pltpu.SemaphoreType.DMA((N_STREAMS, S)),
            pltpu.SemaphoreType.DMA((N_STREAMS, S)),
            pltpu.SemaphoreType.REGULAR((N_STREAMS,)),
        ],
        compiler_params=pltpu.CompilerParams(collective_id=0),
    )(x, w_mat, scale_x, scale_w)


# device time: 180936 ns/iter; 2.3310x vs baseline; 1.0001x over previous
import jax
import jax.numpy as jnp
from jax import lax
from jax.experimental import pallas as pl
from jax.experimental.pallas import tpu as pltpu

N_DEV = 16
N_STREAMS = 4
S = 3


def kernel(x, w_mat, scale_x, scale_w):
    m_glob, k_loc = x.shape
    _, n = w_mat.shape
    m_chunk = m_glob // N_DEV
    n_half = n // 2
    n_sub = n // N_STREAMS

    def body(x_ref, w_ref, sx_ref, sw_ref, out_ref,
             buf_ref, w_bf16_ref, tmp_ref, send_sems, recv_sems, credit_sems):
        my = lax.axis_index("i")
        left = lax.rem(my - 1 + N_DEV, N_DEV)
        right = lax.rem(my + 1, N_DEV)

        peer_out = [right, right, left, left]
        peer_in = [left, left, right, right]

        barrier = pltpu.get_barrier_semaphore()
        for nbr in (left, right):
            pl.semaphore_signal(
                barrier, inc=1,
                device_id=(nbr,), device_id_type=pl.DeviceIdType.MESH,
            )
        pl.semaphore_wait(barrier, 2)

        w_bf16_ref[...] = w_ref[...].astype(jnp.bfloat16)

        def dot_chunk(idx, col0, ncol):
            xa = x_ref[pl.ds(idx * m_chunk, m_chunk), :].astype(jnp.bfloat16)
            return lax.dot_general(
                xa, w_bf16_ref[:, col0:col0 + ncol], (((1,), (0,)), ((), ())),
                preferred_element_type=jnp.float32,
            )

        def mod(v):
            return lax.rem(v + 2 * N_DEV, N_DEV)

        init_idx = [mod(my - 1), mod(my - 1), mod(my + 1), mod(my + 1)]

        order = (0, 2, 1, 3)
        alpha = sx_ref[0] * sw_ref[0]

        def make_rdma(k, h):
            return pltpu.make_async_remote_copy(
                src_ref=buf_ref.at[k, h % S],
                dst_ref=buf_ref.at[k, (h + 1) % S],
                send_sem=send_sems.at[k, h % S],
                recv_sem=recv_sems.at[k, (h + 1) % S],
                device_id=(peer_out[k],),
                device_id_type=pl.DeviceIdType.MESH,
            )

        rdmas = [make_rdma(k, 0) for k in range(N_STREAMS)]
        for k in order:
            buf_ref[k, 0] = dot_chunk(init_idx[k], k * n_sub, n_sub)
            rdmas[k].start()

        for h in range(N_DEV - 1):
            r = (h + 1) % S

            idx_r = mod(my - 2 - h)
            idx_l = mod(my + 2 + h)
            tmp_ref[:, :n_half] = dot_chunk(idx_r, 0, n_half)
            tmp_ref[:, n_half:] = dot_chunk(idx_l, n_half, n_half)

            prev = list(rdmas)
            for k in order:
                cols = pl.ds(k * n_sub, n_sub)
                prev[k].wait_recv()
                if h < N_DEV - 2:
                    buf_ref[k, r] = buf_ref[k, r] + tmp_ref[:, cols]
                    if h + 1 >= S - 1:
                        pl.semaphore_wait(credit_sems.at[k], 1)
                    rdmas[k] = make_rdma(k, h + 1)
                    rdmas[k].start()
                else:
                    y = (buf_ref[k, r] + tmp_ref[:, cols]) * alpha
                    yc = jnp.clip(y, -60.0, 60.0)
                    out_ref[:, cols] = y / (1.0 + jnp.exp(-yc))
            for k in order:
                prev[k].wait_send()
                if h < N_DEV - S:
                    pl.semaphore_signal(
                        credit_sems.at[k], inc=1,
                        device_id=(peer_in[k],),
                        device_id_type=pl.DeviceIdType.MESH,
                    )

    return pl.pallas_call(
        body,
        out_shape=jax.ShapeDtypeStruct((m_chunk, n), jnp.float32),
        in_specs=[
            pl.BlockSpec(memory_space=pltpu.VMEM),
            pl.BlockSpec(memory_space=pltpu.VMEM),
            pl.BlockSpec(memory_space=pltpu.SMEM),
            pl.BlockSpec(memory_space=pltpu.SMEM),
        ],
        out_specs=pl.BlockSpec(memory_space=pltpu.VMEM),
        scratch_shapes=[
            pltpu.VMEM((N_STREAMS, S, m_chunk, n_sub), jnp.float32),
            pltpu.VMEM((k_loc, n), jnp.bfloat16),
            pltpu.VMEM((m_chunk, n), jnp.float32),
            pltpu.SemaphoreType.DMA((N_STREAMS, S)),
            pltpu.SemaphoreType.DMA((N_STREAMS, S)),
            pltpu.SemaphoreType.REGULAR((N_STREAMS,)),
        ],
        compiler_params=pltpu.CompilerParams(collective_id=0),
    )(x, w_mat, scale_x, scale_w)
